# baseline (speedup 1.0000x reference)
import functools

import jax
import jax.numpy as jnp
from jax import lax
from jax.experimental import pallas as pl
from jax.experimental.pallas import tpu as pltpu
from jax.experimental.pallas import tpu_sc as plsc

N = 10000
E = 320000
NP = 10112
NC, NS = 2, 16
NW = NC * NS
EPT = 10240
EP = EPT * NW
NB = EPT // 128
STRIPE = NP // NS

@functools.cache
def _sc_kernels():
    mesh = plsc.VectorSubcoreMesh(core_axis_name="c", subcore_axis_name="s",
                                  num_cores=NC, num_subcores=NS)

    @functools.partial(
        pl.kernel,
        out_type=jax.ShapeDtypeStruct((NW, NP), jnp.float32),
        mesh=mesh,
        scratch_types=[
            pltpu.VMEM((EPT,), jnp.int32),
            pltpu.VMEM((NP,), jnp.float32),
        ],
        compiler_params=pltpu.CompilerParams(needs_layout_passes=False),
    )
    def deg_sc(dst_hbm, deg_out, dst_v, cnt_v):
        cid = lax.axis_index("c")
        sid = lax.axis_index("s")
        wid = sid * NC + cid
        pltpu.sync_copy(dst_hbm.at[wid], dst_v)

        @pl.loop(0, NP // 16)
        def _zero(i):
            cnt_v[pl.ds(i * 16, 16)] = jnp.zeros((16,), jnp.float32)

        ones = jnp.ones((16,), jnp.float32)

        @pl.loop(0, EPT // 16)
        def _count(i):
            idx = dst_v[pl.ds(i * 16, 16)]
            plsc.addupdate_scatter(cnt_v, [idx], ones)

        pltpu.sync_copy(cnt_v, deg_out.at[wid])

    @functools.partial(
        pl.kernel,
        out_type=jax.ShapeDtypeStruct((NC, NP, 128), jnp.float32),
        mesh=mesh,
        scratch_types=[
            pltpu.VMEM((NB, 128), jnp.int32),
            pltpu.VMEM((NB, 128), jnp.int32),
            pltpu.VMEM((128, 128), jnp.float32),
            pltpu.VMEM_SHARED((NP, 128), jnp.float32),
            pltpu.SemaphoreType.DMA,
        ],
    )
    def agg_sc(src_hbm, dst_hbm, g_hbm, zeros_hbm, out_hbm,
               src_v, dst_v, rows_v, acc, sem):
        cid = lax.axis_index("c")
        sid = lax.axis_index("s")
        wid = sid * NC + cid

        base = sid * STRIPE
        for k in range(STRIPE // 128):
            pltpu.sync_copy(zeros_hbm, acc.at[pl.ds(base + k * 128, 128)])
        rem = STRIPE % 128
        if rem:
            pltpu.sync_copy(zeros_hbm.at[pl.ds(0, rem)],
                            acc.at[pl.ds(base + (STRIPE // 128) * 128, rem)])

        pltpu.sync_copy(src_hbm.at[wid], src_v)
        pltpu.sync_copy(dst_hbm.at[wid], dst_v)
        plsc.subcore_barrier()

        @pl.loop(0, NB)
        def _edges(j):
            pltpu.async_copy(g_hbm.at[src_v.at[j]], rows_v, sem).wait()
            pltpu.sync_copy(rows_v, acc.at[dst_v.at[j]], add=True)

        plsc.subcore_barrier()
        pltpu.sync_copy(acc.at[pl.ds(base, STRIPE)],
                        out_hbm.at[cid, pl.ds(base, STRIPE)])

    return deg_sc, agg_sc


def _deg_sc(dst2):
    return _sc_kernels()[0](dst2)


def _agg_sc(src3, dst3, g, zeros128):
    return _sc_kernels()[1](src3, dst3, g, zeros128)


_R = NP // 8


def _dot(a, w):
    return lax.dot_general(a, w, (((1,), (0,)), ((), ())),
                           precision=lax.Precision.HIGHEST,
                           preferred_element_type=jnp.float32)


def _dis_body(deg_ref, x_ref, dis_ref, g1_ref, *, blk):
    i = pl.program_id(0)
    deg = jnp.sum(deg_ref[...], axis=1, keepdims=True) + 1.0
    dis = lax.rsqrt(deg)
    row = i * blk + lax.broadcasted_iota(jnp.int32, (blk, 1), 0)
    dis = jnp.where(row < N, dis, 0.0)
    dis_ref[...] = jnp.broadcast_to(dis, (blk, 128))
    g1_ref[...] = dis * x_ref[...]


def _dis_g1(deg_parts, x_pad):
    return pl.pallas_call(
        functools.partial(_dis_body, blk=_R),
        grid=(NP // _R,),
        in_specs=[
            pl.BlockSpec((_R, NW), lambda i: (i, 0)),
            pl.BlockSpec((_R, 128), lambda i: (i, 0)),
        ],
        out_specs=[
            pl.BlockSpec((_R, 128), lambda i: (i, 0)),
            pl.BlockSpec((_R, 128), lambda i: (i, 0)),
        ],
        out_shape=[
            jax.ShapeDtypeStruct((NP, 128), jnp.float32),
            jax.ShapeDtypeStruct((NP, 128), jnp.float32),
        ],
    )(deg_parts, x_pad)


def _l1_body(s_ref, g_ref, dis_ref, w_ref, b_ref, out_ref):
    dis = dis_ref[...]
    a = dis * (s_ref[0] + s_ref[1] + g_ref[...])
    h = jnp.maximum(_dot(a, w_ref[...]) + b_ref[...], 0.0)
    d1 = dis[:, 0:1]
    out_ref[0] = d1 * h[:, :128]
    out_ref[1] = d1 * h[:, 128:]


def _layer1(s1, g1, dis_b, W1, b1):
    return pl.pallas_call(
        _l1_body,
        grid=(NP // _R,),
        in_specs=[
            pl.BlockSpec((2, _R, 128), lambda i: (0, i, 0)),
            pl.BlockSpec((_R, 128), lambda i: (i, 0)),
            pl.BlockSpec((_R, 128), lambda i: (i, 0)),
            pl.BlockSpec((128, 256), lambda i: (0, 0)),
            pl.BlockSpec((1, 256), lambda i: (0, 0)),
        ],
        out_specs=pl.BlockSpec((2, _R, 128), lambda i: (0, i, 0)),
        out_shape=jax.ShapeDtypeStruct((2, NP, 128), jnp.float32),
    )(s1, g1, dis_b, W1, b1)


def _mid_body(sa_ref, sb_ref, g_ref, dis_ref, w_ref, b_ref, out_ref, *,
              w2_ref=None):
    dis = dis_ref[...]
    t0 = dis * (sa_ref[0] + sa_ref[1] + g_ref[0])
    t1 = dis * (sb_ref[0] + sb_ref[1] + g_ref[1])
    a = jnp.concatenate([t0, t1], axis=1)
    h = jnp.maximum(_dot(a, w_ref[...]) + b_ref[...], 0.0)
    d1 = dis[:, 0:1]
    if w2_ref is None:
        out_ref[0] = d1 * h[:, :128]
        out_ref[1] = d1 * h[:, 128:]
    else:
        out_ref[...] = d1 * _dot(h, w2_ref[...])


def _layer_mid(sa, sb, g, dis_b, W, b):
    return pl.pallas_call(
        _mid_body,
        grid=(NP // _R,),
        in_specs=[
            pl.BlockSpec((2, _R, 128), lambda i: (0, i, 0)),
            pl.BlockSpec((2, _R, 128), lambda i: (0, i, 0)),
            pl.BlockSpec((2, _R, 128), lambda i: (0, i, 0)),
            pl.BlockSpec((_R, 128), lambda i: (i, 0)),
            pl.BlockSpec((256, 256), lambda i: (0, 0)),
            pl.BlockSpec((1, 256), lambda i: (0, 0)),
        ],
        out_specs=pl.BlockSpec((2, _R, 128), lambda i: (0, i, 0)),
        out_shape=jax.ShapeDtypeStruct((2, NP, 128), jnp.float32),
    )(sa, sb, g, dis_b, W, b)


def _l3_body(sa_ref, sb_ref, g_ref, dis_ref, w_ref, b_ref, w2_ref, out_ref):
    _mid_body(sa_ref, sb_ref, g_ref, dis_ref, w_ref, b_ref, out_ref,
              w2_ref=w2_ref)


def _layer3(sa, sb, g, dis_b, W3, b3, W4):
    return pl.pallas_call(
        _l3_body,
        grid=(NP // _R,),
        in_specs=[
            pl.BlockSpec((2, _R, 128), lambda i: (0, i, 0)),
            pl.BlockSpec((2, _R, 128), lambda i: (0, i, 0)),
            pl.BlockSpec((2, _R, 128), lambda i: (0, i, 0)),
            pl.BlockSpec((_R, 128), lambda i: (i, 0)),
            pl.BlockSpec((256, 256), lambda i: (0, 0)),
            pl.BlockSpec((1, 256), lambda i: (0, 0)),
            pl.BlockSpec((256, 128), lambda i: (0, 0)),
        ],
        out_specs=pl.BlockSpec((_R, 128), lambda i: (i, 0)),
        out_shape=jax.ShapeDtypeStruct((NP, 128), jnp.float32),
    )(sa, sb, g, dis_b, W3, b3, W4)


def _l4_body(s_ref, g_ref, dis_ref, b_ref, out_ref):
    z = dis_ref[...] * (s_ref[0] + s_ref[1] + g_ref[...]) + b_ref[...]
    m = jnp.max(z, axis=1, keepdims=True)
    zm = z - m
    out_ref[...] = zm - jnp.log(jnp.sum(jnp.exp(zm), axis=1, keepdims=True))


def _layer4(s4, g4, dis_b, b4):
    blk = 400
    return pl.pallas_call(
        _l4_body,
        grid=(N // blk,),
        in_specs=[
            pl.BlockSpec((2, blk, 128), lambda i: (0, i, 0)),
            pl.BlockSpec((blk, 128), lambda i: (i, 0)),
            pl.BlockSpec((blk, 128), lambda i: (i, 0)),
            pl.BlockSpec((1, 128), lambda i: (0, 0)),
        ],
        out_specs=pl.BlockSpec((blk, 128), lambda i: (i, 0)),
        out_shape=jax.ShapeDtypeStruct((N, 128), jnp.float32),
    )(s4, g4, dis_b, b4)


def kernel(x, edge_index, W1, b1, W2, b2, W3, b3, W4, b4):
    src = edge_index[0].astype(jnp.int32)
    dst = edge_index[1].astype(jnp.int32)
    pad = jnp.full((EP - E,), N, jnp.int32)
    src3 = jnp.concatenate([src, pad]).reshape(NW, NB, 128)
    dst3 = jnp.concatenate([dst, pad]).reshape(NW, NB, 128)
    dst2 = dst3.reshape(NW, EPT)
    x_pad = jnp.pad(x, ((0, NP - N), (0, 0)))
    zeros128 = jnp.zeros((128, 128), jnp.float32)

    agg = lambda g: _agg_sc(src3, dst3, g, zeros128)

    b1r, b2r, b3r, b4r = (b.reshape(1, -1) for b in (b1, b2, b3, b4))
    deg_parts = _deg_sc(dst2)
    dis_b, g1 = _dis_g1(deg_parts.T, x_pad)
    s1 = agg(g1)
    g2 = _layer1(s1, g1, dis_b, W1, b1r)
    s2a, s2b = agg(g2[0]), agg(g2[1])
    g3 = _layer_mid(s2a, s2b, g2, dis_b, W2, b2r)
    s3a, s3b = agg(g3[0]), agg(g3[1])
    g4 = _layer3(s3a, s3b, g3, dis_b, W3, b3r, W4)
    s4 = agg(g4)
    return _layer4(s4, g4, dis_b, b4r)


# speedup vs baseline: 2.5180x; 2.5180x over previous
import functools

import jax
import jax.numpy as jnp
from jax import lax
from jax.experimental import pallas as pl
from jax.experimental.pallas import tpu as pltpu
from jax.experimental.pallas import tpu_sc as plsc

N = 10000
E = 320000
NP = 10112
NC, NS = 2, 16
NW = NC * NS
EPT = 10240
EP = EPT * NW
NB = EPT // 128
STRIPE = NP // NS

@functools.cache
def _sc_kernels():
    mesh = plsc.VectorSubcoreMesh(core_axis_name="c", subcore_axis_name="s",
                                  num_cores=NC, num_subcores=NS)

    @functools.partial(
        pl.kernel,
        out_type=jax.ShapeDtypeStruct((NW, NP), jnp.float32),
        mesh=mesh,
        scratch_types=[
            pltpu.VMEM((EPT,), jnp.int32),
            pltpu.VMEM((NP,), jnp.float32),
        ],
        compiler_params=pltpu.CompilerParams(needs_layout_passes=False),
    )
    def deg_sc(dst_hbm, deg_out, dst_v, cnt_v):
        cid = lax.axis_index("c")
        sid = lax.axis_index("s")
        wid = sid * NC + cid
        pltpu.sync_copy(dst_hbm.at[wid], dst_v)

        @pl.loop(0, NP // 16)
        def _zero(i):
            cnt_v[pl.ds(i * 16, 16)] = jnp.zeros((16,), jnp.float32)

        ones = jnp.ones((16,), jnp.float32)

        @pl.loop(0, EPT // 16)
        def _count(i):
            idx = dst_v[pl.ds(i * 16, 16)]
            plsc.addupdate_scatter(cnt_v, [idx], ones)

        pltpu.sync_copy(cnt_v, deg_out.at[wid])

    @functools.partial(
        pl.kernel,
        out_type=jax.ShapeDtypeStruct((NC, NP, 128), jnp.float32),
        mesh=mesh,
        scratch_types=[
            pltpu.VMEM((NB, 128), jnp.int32),
            pltpu.VMEM((NB, 128), jnp.int32),
            pltpu.VMEM((128, 128), jnp.float32),
            pltpu.VMEM_SHARED((NP, 128), jnp.float32),
            pltpu.SemaphoreType.DMA,
        ],
    )
    def agg_sc(src_hbm, dst_hbm, g_hbm, zeros_hbm, out_hbm,
               src_v, dst_v, rows_v, acc, sem):
        cid = lax.axis_index("c")
        sid = lax.axis_index("s")
        wid = sid * NC + cid

        base = sid * STRIPE
        for k in range(STRIPE // 128):
            pltpu.sync_copy(zeros_hbm, acc.at[pl.ds(base + k * 128, 128)])
        rem = STRIPE % 128
        if rem:
            pltpu.sync_copy(zeros_hbm.at[pl.ds(0, rem)],
                            acc.at[pl.ds(base + (STRIPE // 128) * 128, rem)])

        pltpu.sync_copy(src_hbm.at[wid], src_v)
        pltpu.sync_copy(dst_hbm.at[wid], dst_v)
        plsc.subcore_barrier()

        @pl.loop(0, NB)
        def _edges(j):
            pltpu.async_copy(g_hbm.at[src_v.at[j]], rows_v, sem).wait()
            pltpu.sync_copy(rows_v, acc.at[dst_v.at[j]], add=True)

        plsc.subcore_barrier()
        pltpu.sync_copy(acc.at[pl.ds(base, STRIPE)],
                        out_hbm.at[cid, pl.ds(base, STRIPE)])

    return deg_sc, agg_sc


def _deg_sc(dst2):
    return _sc_kernels()[0](dst2)


def _agg_sc(src3, dst3, g, zeros128):
    return _sc_kernels()[1](src3, dst3, g, zeros128)


_R = NP // 8


def _dot(a, w):
    return lax.dot_general(a, w, (((1,), (0,)), ((), ())),
                           precision=lax.Precision.HIGHEST,
                           preferred_element_type=jnp.float32)


def _dis_body(deg_ref, x_ref, dis_ref, g1_ref, *, blk):
    i = pl.program_id(0)
    deg = jnp.sum(deg_ref[...], axis=1, keepdims=True) + 1.0
    dis = lax.rsqrt(deg)
    row = i * blk + lax.broadcasted_iota(jnp.int32, (blk, 1), 0)
    dis = jnp.where(row < N, dis, 0.0)
    dis_ref[...] = jnp.broadcast_to(dis, (blk, 128))
    g1_ref[...] = dis * x_ref[...]


def _dis_g1(deg_parts, x_pad):
    return pl.pallas_call(
        functools.partial(_dis_body, blk=_R),
        grid=(NP // _R,),
        in_specs=[
            pl.BlockSpec((_R, NW), lambda i: (i, 0)),
            pl.BlockSpec((_R, 128), lambda i: (i, 0)),
        ],
        out_specs=[
            pl.BlockSpec((_R, 128), lambda i: (i, 0)),
            pl.BlockSpec((_R, 128), lambda i: (i, 0)),
        ],
        out_shape=[
            jax.ShapeDtypeStruct((NP, 128), jnp.float32),
            jax.ShapeDtypeStruct((NP, 128), jnp.float32),
        ],
    )(deg_parts, x_pad)


def _l1_body(s_ref, g_ref, dis_ref, w_ref, b_ref, out_ref):
    dis = dis_ref[...]
    a = dis * (s_ref[0] + s_ref[1] + g_ref[...])
    h = jnp.maximum(_dot(a, w_ref[...]) + b_ref[...], 0.0)
    d1 = dis[:, 0:1]
    out_ref[0] = d1 * h[:, :128]
    out_ref[1] = d1 * h[:, 128:]


def _layer1(s1, g1, dis_b, W1, b1):
    return pl.pallas_call(
        _l1_body,
        grid=(NP // _R,),
        in_specs=[
            pl.BlockSpec((2, _R, 128), lambda i: (0, i, 0)),
            pl.BlockSpec((_R, 128), lambda i: (i, 0)),
            pl.BlockSpec((_R, 128), lambda i: (i, 0)),
            pl.BlockSpec((128, 256), lambda i: (0, 0)),
            pl.BlockSpec((1, 256), lambda i: (0, 0)),
        ],
        out_specs=pl.BlockSpec((2, _R, 128), lambda i: (0, i, 0)),
        out_shape=jax.ShapeDtypeStruct((2, NP, 128), jnp.float32),
    )(s1, g1, dis_b, W1, b1)


def _mid_body(sa_ref, sb_ref, g_ref, dis_ref, w_ref, b_ref, out_ref, *,
              w2_ref=None):
    dis = dis_ref[...]
    t0 = dis * (sa_ref[0] + sa_ref[1] + g_ref[0])
    t1 = dis * (sb_ref[0] + sb_ref[1] + g_ref[1])
    a = jnp.concatenate([t0, t1], axis=1)
    h = jnp.maximum(_dot(a, w_ref[...]) + b_ref[...], 0.0)
    d1 = dis[:, 0:1]
    if w2_ref is None:
        out_ref[0] = d1 * h[:, :128]
        out_ref[1] = d1 * h[:, 128:]
    else:
        out_ref[...] = d1 * _dot(h, w2_ref[...])


def _layer_mid(sa, sb, g, dis_b, W, b):
    return pl.pallas_call(
        _mid_body,
        grid=(NP // _R,),
        in_specs=[
            pl.BlockSpec((2, _R, 128), lambda i: (0, i, 0)),
            pl.BlockSpec((2, _R, 128), lambda i: (0, i, 0)),
            pl.BlockSpec((2, _R, 128), lambda i: (0, i, 0)),
            pl.BlockSpec((_R, 128), lambda i: (i, 0)),
            pl.BlockSpec((256, 256), lambda i: (0, 0)),
            pl.BlockSpec((1, 256), lambda i: (0, 0)),
        ],
        out_specs=pl.BlockSpec((2, _R, 128), lambda i: (0, i, 0)),
        out_shape=jax.ShapeDtypeStruct((2, NP, 128), jnp.float32),
    )(sa, sb, g, dis_b, W, b)


def _l3_body(sa_ref, sb_ref, g_ref, dis_ref, w_ref, b_ref, w2_ref, out_ref):
    _mid_body(sa_ref, sb_ref, g_ref, dis_ref, w_ref, b_ref, out_ref,
              w2_ref=w2_ref)


def _layer3(sa, sb, g, dis_b, W3, b3, W4):
    return pl.pallas_call(
        _l3_body,
        grid=(NP // _R,),
        in_specs=[
            pl.BlockSpec((2, _R, 128), lambda i: (0, i, 0)),
            pl.BlockSpec((2, _R, 128), lambda i: (0, i, 0)),
            pl.BlockSpec((2, _R, 128), lambda i: (0, i, 0)),
            pl.BlockSpec((_R, 128), lambda i: (i, 0)),
            pl.BlockSpec((256, 256), lambda i: (0, 0)),
            pl.BlockSpec((1, 256), lambda i: (0, 0)),
            pl.BlockSpec((256, 128), lambda i: (0, 0)),
        ],
        out_specs=pl.BlockSpec((_R, 128), lambda i: (i, 0)),
        out_shape=jax.ShapeDtypeStruct((NP, 128), jnp.float32),
    )(sa, sb, g, dis_b, W3, b3, W4)


def _l4_body(s_ref, g_ref, dis_ref, b_ref, out_ref):
    z = dis_ref[...] * (s_ref[0] + s_ref[1] + g_ref[...]) + b_ref[...]
    m = jnp.max(z, axis=1, keepdims=True)
    zm = z - m
    out_ref[...] = zm - jnp.log(jnp.sum(jnp.exp(zm), axis=1, keepdims=True))


def _layer4(s4, g4, dis_b, b4):
    blk = 400
    return pl.pallas_call(
        _l4_body,
        grid=(N // blk,),
        in_specs=[
            pl.BlockSpec((2, blk, 128), lambda i: (0, i, 0)),
            pl.BlockSpec((blk, 128), lambda i: (i, 0)),
            pl.BlockSpec((blk, 128), lambda i: (i, 0)),
            pl.BlockSpec((1, 128), lambda i: (0, 0)),
        ],
        out_specs=pl.BlockSpec((blk, 128), lambda i: (i, 0)),
        out_shape=jax.ShapeDtypeStruct((N, 128), jnp.float32),
    )(s4, g4, dis_b, b4)


def kernel(x, edge_index, W1, b1, W2, b2, W3, b3, W4, b4):
    src = edge_index[0].astype(jnp.int32)
    dst = edge_index[1].astype(jnp.int32)
    pad = N + jnp.arange(EP - E, dtype=jnp.int32) % (NP - N)
    src3 = jnp.concatenate([src, pad]).reshape(NW, NB, 128)
    dst3 = jnp.concatenate([dst, pad]).reshape(NW, NB, 128)
    dst2 = dst3.reshape(NW, EPT)
    x_pad = jnp.pad(x, ((0, NP - N), (0, 0)))
    zeros128 = jnp.zeros((128, 128), jnp.float32)

    agg = lambda g: _agg_sc(src3, dst3, g, zeros128)

    b1r, b2r, b3r, b4r = (b.reshape(1, -1) for b in (b1, b2, b3, b4))
    deg_parts = _deg_sc(dst2)
    dis_b, g1 = _dis_g1(deg_parts.T, x_pad)
    s1 = agg(g1)
    g2 = _layer1(s1, g1, dis_b, W1, b1r)
    s2a, s2b = agg(g2[0]), agg(g2[1])
    g3 = _layer_mid(s2a, s2b, g2, dis_b, W2, b2r)
    s3a, s3b = agg(g3[0]), agg(g3[1])
    g4 = _layer3(s3a, s3b, g3, dis_b, W3, b3r, W4)
    s4 = agg(g4)
    return _layer4(s4, g4, dis_b, b4r)


# speedup vs baseline: 3.6838x; 1.4630x over previous
import functools

import jax
import jax.numpy as jnp
from jax import lax
from jax.experimental import pallas as pl
from jax.experimental.pallas import tpu as pltpu
from jax.experimental.pallas import tpu_sc as plsc

N = 10000
E = 320000
NP = 10112
NC, NS = 2, 16
NW = NC * NS
EPT = 10240
EP = EPT * NW
NB = EPT // 128
GB = 8
G = NB // GB
STRIPE = NP // NS

@functools.cache
def _sc_kernels():
    mesh = plsc.VectorSubcoreMesh(core_axis_name="c", subcore_axis_name="s",
                                  num_cores=NC, num_subcores=NS)

    @functools.partial(
        pl.kernel,
        out_type=jax.ShapeDtypeStruct((NW, NP), jnp.float32),
        mesh=mesh,
        scratch_types=[
            pltpu.VMEM((EPT,), jnp.int32),
            pltpu.VMEM((NP,), jnp.float32),
        ],
        compiler_params=pltpu.CompilerParams(needs_layout_passes=False),
    )
    def deg_sc(dst_hbm, deg_out, dst_v, cnt_v):
        cid = lax.axis_index("c")
        sid = lax.axis_index("s")
        wid = sid * NC + cid
        pltpu.sync_copy(dst_hbm.at[wid], dst_v)

        @pl.loop(0, NP // 16)
        def _zero(i):
            cnt_v[pl.ds(i * 16, 16)] = jnp.zeros((16,), jnp.float32)

        ones = jnp.ones((16,), jnp.float32)

        @pl.loop(0, EPT // 16)
        def _count(i):
            idx = dst_v[pl.ds(i * 16, 16)]
            plsc.addupdate_scatter(cnt_v, [idx], ones)

        pltpu.sync_copy(cnt_v, deg_out.at[wid])

    @functools.partial(
        pl.kernel,
        out_type=jax.ShapeDtypeStruct((NC, NP, 128), jnp.float32),
        mesh=mesh,
        scratch_types=[
            pltpu.VMEM((NB, 128), jnp.int32),
            pltpu.VMEM((2, GB, 128), jnp.int32),
            pltpu.VMEM((2, 128, 128), jnp.float32),
            pltpu.VMEM_SHARED((NP, 128), jnp.float32),
            pltpu.SemaphoreType.DMA,
            pltpu.SemaphoreType.DMA,
            pltpu.SemaphoreType.DMA,
            pltpu.SemaphoreType.DMA,
        ],
    )
    def agg_sc(src_hbm, dst_hbm, g_hbm, zeros_hbm, out_hbm,
               src_v, dst_c, rows_v, acc, gs0, gs1, is0, is1):
        cid = lax.axis_index("c")
        sid = lax.axis_index("s")
        wid = sid * NC + cid
        gsem = (gs0, gs1)
        isem = (is0, is1)

        base = sid * STRIPE
        for k in range(STRIPE // 128):
            pltpu.sync_copy(zeros_hbm, acc.at[pl.ds(base + k * 128, 128)])
        rem = STRIPE % 128
        if rem:
            pltpu.sync_copy(zeros_hbm.at[pl.ds(0, rem)],
                            acc.at[pl.ds(base + (STRIPE // 128) * 128, rem)])

        pltpu.sync_copy(src_hbm.at[wid], src_v)
        pltpu.sync_copy(dst_hbm.at[wid, pl.ds(0, GB)], dst_c.at[0])
        plsc.subcore_barrier()
        pltpu.async_copy(g_hbm.at[src_v.at[0]], rows_v.at[0], gsem[0])

        @pl.loop(0, G)
        def _group(g):
            for q in range(2):
                @pl.when(lax.rem(g, 2) == q)
                def _():
                    @pl.when(g + 1 < G)
                    def _():
                        pltpu.async_copy(
                            dst_hbm.at[wid, pl.ds((g + 1) * GB, GB)],
                            dst_c.at[1 - q], isem[1 - q])
                    for jj in range(GB):
                        p = jj % 2
                        j = g * GB + jj

                        @pl.when(j + 1 < NB)
                        def _():
                            pltpu.async_copy(g_hbm.at[src_v.at[j + 1]],
                                             rows_v.at[1 - p], gsem[1 - p])
                        pltpu.make_async_copy(
                            g_hbm.at[src_v.at[j]], rows_v.at[p],
                            gsem[p]).wait()
                        pltpu.sync_copy(rows_v.at[p],
                                        acc.at[dst_c.at[q, jj]], add=True)

                    @pl.when(g + 1 < G)
                    def _():
                        pltpu.make_async_copy(
                            dst_hbm.at[wid, pl.ds(0, GB)],
                            dst_c.at[1 - q], isem[1 - q]).wait()

        plsc.subcore_barrier()
        pltpu.sync_copy(acc.at[pl.ds(base, STRIPE)],
                        out_hbm.at[cid, pl.ds(base, STRIPE)])

    return deg_sc, agg_sc


def _deg_sc(dst2):
    return _sc_kernels()[0](dst2)


def _agg_sc(src3, dst3, g, zeros128):
    return _sc_kernels()[1](src3, dst3, g, zeros128)


_R = NP // 8


def _dot(a, w):
    return lax.dot_general(a, w, (((1,), (0,)), ((), ())),
                           precision=lax.Precision.HIGHEST,
                           preferred_element_type=jnp.float32)


def _dis_body(deg_ref, x_ref, dis_ref, g1_ref, *, blk):
    i = pl.program_id(0)
    deg = jnp.sum(deg_ref[...], axis=1, keepdims=True) + 1.0
    dis = lax.rsqrt(deg)
    row = i * blk + lax.broadcasted_iota(jnp.int32, (blk, 1), 0)
    dis = jnp.where(row < N, dis, 0.0)
    dis_ref[...] = jnp.broadcast_to(dis, (blk, 128))
    g1_ref[...] = dis * x_ref[...]


def _dis_g1(deg_parts, x_pad):
    return pl.pallas_call(
        functools.partial(_dis_body, blk=_R),
        grid=(NP // _R,),
        in_specs=[
            pl.BlockSpec((_R, NW), lambda i: (i, 0)),
            pl.BlockSpec((_R, 128), lambda i: (i, 0)),
        ],
        out_specs=[
            pl.BlockSpec((_R, 128), lambda i: (i, 0)),
            pl.BlockSpec((_R, 128), lambda i: (i, 0)),
        ],
        out_shape=[
            jax.ShapeDtypeStruct((NP, 128), jnp.float32),
            jax.ShapeDtypeStruct((NP, 128), jnp.float32),
        ],
    )(deg_parts, x_pad)


def _l1_body(s_ref, g_ref, dis_ref, w_ref, b_ref, out_ref):
    dis = dis_ref[...]
    a = dis * (s_ref[0] + s_ref[1] + g_ref[...])
    h = jnp.maximum(_dot(a, w_ref[...]) + b_ref[...], 0.0)
    d1 = dis[:, 0:1]
    out_ref[0] = d1 * h[:, :128]
    out_ref[1] = d1 * h[:, 128:]


def _layer1(s1, g1, dis_b, W1, b1):
    return pl.pallas_call(
        _l1_body,
        grid=(NP // _R,),
        in_specs=[
            pl.BlockSpec((2, _R, 128), lambda i: (0, i, 0)),
            pl.BlockSpec((_R, 128), lambda i: (i, 0)),
            pl.BlockSpec((_R, 128), lambda i: (i, 0)),
            pl.BlockSpec((128, 256), lambda i: (0, 0)),
            pl.BlockSpec((1, 256), lambda i: (0, 0)),
        ],
        out_specs=pl.BlockSpec((2, _R, 128), lambda i: (0, i, 0)),
        out_shape=jax.ShapeDtypeStruct((2, NP, 128), jnp.float32),
    )(s1, g1, dis_b, W1, b1)


def _mid_body(sa_ref, sb_ref, g_ref, dis_ref, w_ref, b_ref, out_ref, *,
              w2_ref=None):
    dis = dis_ref[...]
    t0 = dis * (sa_ref[0] + sa_ref[1] + g_ref[0])
    t1 = dis * (sb_ref[0] + sb_ref[1] + g_ref[1])
    a = jnp.concatenate([t0, t1], axis=1)
    h = jnp.maximum(_dot(a, w_ref[...]) + b_ref[...], 0.0)
    d1 = dis[:, 0:1]
    if w2_ref is None:
        out_ref[0] = d1 * h[:, :128]
        out_ref[1] = d1 * h[:, 128:]
    else:
        out_ref[...] = d1 * _dot(h, w2_ref[...])


def _layer_mid(sa, sb, g, dis_b, W, b):
    return pl.pallas_call(
        _mid_body,
        grid=(NP // _R,),
        in_specs=[
            pl.BlockSpec((2, _R, 128), lambda i: (0, i, 0)),
            pl.BlockSpec((2, _R, 128), lambda i: (0, i, 0)),
            pl.BlockSpec((2, _R, 128), lambda i: (0, i, 0)),
            pl.BlockSpec((_R, 128), lambda i: (i, 0)),
            pl.BlockSpec((256, 256), lambda i: (0, 0)),
            pl.BlockSpec((1, 256), lambda i: (0, 0)),
        ],
        out_specs=pl.BlockSpec((2, _R, 128), lambda i: (0, i, 0)),
        out_shape=jax.ShapeDtypeStruct((2, NP, 128), jnp.float32),
    )(sa, sb, g, dis_b, W, b)


def _l3_body(sa_ref, sb_ref, g_ref, dis_ref, w_ref, b_ref, w2_ref, out_ref):
    _mid_body(sa_ref, sb_ref, g_ref, dis_ref, w_ref, b_ref, out_ref,
              w2_ref=w2_ref)


def _layer3(sa, sb, g, dis_b, W3, b3, W4):
    return pl.pallas_call(
        _l3_body,
        grid=(NP // _R,),
        in_specs=[
            pl.BlockSpec((2, _R, 128), lambda i: (0, i, 0)),
            pl.BlockSpec((2, _R, 128), lambda i: (0, i, 0)),
            pl.BlockSpec((2, _R, 128), lambda i: (0, i, 0)),
            pl.BlockSpec((_R, 128), lambda i: (i, 0)),
            pl.BlockSpec((256, 256), lambda i: (0, 0)),
            pl.BlockSpec((1, 256), lambda i: (0, 0)),
            pl.BlockSpec((256, 128), lambda i: (0, 0)),
        ],
        out_specs=pl.BlockSpec((_R, 128), lambda i: (i, 0)),
        out_shape=jax.ShapeDtypeStruct((NP, 128), jnp.float32),
    )(sa, sb, g, dis_b, W3, b3, W4)


def _l4_body(s_ref, g_ref, dis_ref, b_ref, out_ref):
    z = dis_ref[...] * (s_ref[0] + s_ref[1] + g_ref[...]) + b_ref[...]
    m = jnp.max(z, axis=1, keepdims=True)
    zm = z - m
    out_ref[...] = zm - jnp.log(jnp.sum(jnp.exp(zm), axis=1, keepdims=True))


def _layer4(s4, g4, dis_b, b4):
    blk = 400
    return pl.pallas_call(
        _l4_body,
        grid=(N // blk,),
        in_specs=[
            pl.BlockSpec((2, blk, 128), lambda i: (0, i, 0)),
            pl.BlockSpec((blk, 128), lambda i: (i, 0)),
            pl.BlockSpec((blk, 128), lambda i: (i, 0)),
            pl.BlockSpec((1, 128), lambda i: (0, 0)),
        ],
        out_specs=pl.BlockSpec((blk, 128), lambda i: (i, 0)),
        out_shape=jax.ShapeDtypeStruct((N, 128), jnp.float32),
    )(s4, g4, dis_b, b4)


def kernel(x, edge_index, W1, b1, W2, b2, W3, b3, W4, b4):
    src = edge_index[0].astype(jnp.int32)
    dst = edge_index[1].astype(jnp.int32)
    pad = N + jnp.arange(EP - E, dtype=jnp.int32) % (NP - N)
    src3 = jnp.concatenate([src, pad]).reshape(NW, NB, 128)
    dst3 = jnp.concatenate([dst, pad]).reshape(NW, NB, 128)
    dst2 = dst3.reshape(NW, EPT)
    x_pad = jnp.pad(x, ((0, NP - N), (0, 0)))
    zeros128 = jnp.zeros((128, 128), jnp.float32)

    agg = lambda g: _agg_sc(src3, dst3, g, zeros128)

    b1r, b2r, b3r, b4r = (b.reshape(1, -1) for b in (b1, b2, b3, b4))
    deg_parts = _deg_sc(dst2)
    dis_b, g1 = _dis_g1(deg_parts.T, x_pad)
    s1 = agg(g1)
    g2 = _layer1(s1, g1, dis_b, W1, b1r)
    s2a, s2b = agg(g2[0]), agg(g2[1])
    g3 = _layer_mid(s2a, s2b, g2, dis_b, W2, b2r)
    s3a, s3b = agg(g3[0]), agg(g3[1])
    g4 = _layer3(s3a, s3b, g3, dis_b, W3, b3r, W4)
    s4 = agg(g4)
    return _layer4(s4, g4, dis_b, b4r)


# speedup vs baseline: 3.9835x; 1.0814x over previous
import functools

import jax
import jax.numpy as jnp
from jax import lax
from jax.experimental import pallas as pl
from jax.experimental.pallas import tpu as pltpu
from jax.experimental.pallas import tpu_sc as plsc

N = 10000
E = 320000
NP = 10112
NC, NS = 2, 16
NW = NC * NS
EPT = 10240
EP = EPT * NW
NB = EPT // 128
GB = 8
G = NB // GB
STRIPE = NP // NS

@functools.cache
def _sc_kernels():
    mesh = plsc.VectorSubcoreMesh(core_axis_name="c", subcore_axis_name="s",
                                  num_cores=NC, num_subcores=NS)

    @functools.partial(
        pl.kernel,
        out_type=jax.ShapeDtypeStruct((NW, NP), jnp.float32),
        mesh=mesh,
        scratch_types=[
            pltpu.VMEM((EPT,), jnp.int32),
            pltpu.VMEM((NP,), jnp.float32),
        ],
        compiler_params=pltpu.CompilerParams(needs_layout_passes=False),
    )
    def deg_sc(dst_hbm, deg_out, dst_v, cnt_v):
        cid = lax.axis_index("c")
        sid = lax.axis_index("s")
        wid = sid * NC + cid
        pltpu.sync_copy(dst_hbm.at[wid], dst_v)

        @pl.loop(0, NP // 16)
        def _zero(i):
            cnt_v[pl.ds(i * 16, 16)] = jnp.zeros((16,), jnp.float32)

        ones = jnp.ones((16,), jnp.float32)

        @pl.loop(0, EPT // 16)
        def _count(i):
            idx = dst_v[pl.ds(i * 16, 16)]
            plsc.addupdate_scatter(cnt_v, [idx], ones)

        pltpu.sync_copy(cnt_v, deg_out.at[wid])

    @functools.partial(
        pl.kernel,
        out_type=jax.ShapeDtypeStruct((NC, NP, 128), jnp.float32),
        mesh=mesh,
        scratch_types=[
            pltpu.VMEM((NB, 128), jnp.int32),
            pltpu.VMEM((2, GB, 128), jnp.int32),
            pltpu.VMEM((2, 128, 128), jnp.float32),
            pltpu.VMEM_SHARED((NP, 128), jnp.float32),
            pltpu.SemaphoreType.DMA,
            pltpu.SemaphoreType.DMA,
            pltpu.SemaphoreType.DMA,
            pltpu.SemaphoreType.DMA,
        ],
    )
    def agg_sc(src_hbm, dst_hbm, g_hbm, zeros_hbm, out_hbm,
               src_v, dst_c, rows_v, acc, gs0, gs1, is0, is1):
        cid = lax.axis_index("c")
        sid = lax.axis_index("s")
        wid = sid * NC + cid
        gsem = (gs0, gs1)
        isem = (is0, is1)

        base = sid * STRIPE
        for k in range(STRIPE // 128):
            pltpu.sync_copy(zeros_hbm, acc.at[pl.ds(base + k * 128, 128)])
        rem = STRIPE % 128
        if rem:
            pltpu.sync_copy(zeros_hbm.at[pl.ds(0, rem)],
                            acc.at[pl.ds(base + (STRIPE // 128) * 128, rem)])

        pltpu.sync_copy(src_hbm.at[wid], src_v)
        pltpu.sync_copy(dst_hbm.at[wid, pl.ds(0, GB)], dst_c.at[0])
        plsc.subcore_barrier()
        pltpu.async_copy(g_hbm.at[src_v.at[0]], rows_v.at[0], gsem[0])

        @pl.loop(0, G)
        def _group(g):
            for q in range(2):
                @pl.when(lax.rem(g, 2) == q)
                def _():
                    @pl.when(g + 1 < G)
                    def _():
                        pltpu.async_copy(
                            dst_hbm.at[wid, pl.ds((g + 1) * GB, GB)],
                            dst_c.at[1 - q], isem[1 - q])
                    for jj in range(GB):
                        p = jj % 2
                        j = g * GB + jj

                        @pl.when(j + 1 < NB)
                        def _():
                            pltpu.async_copy(g_hbm.at[src_v.at[j + 1]],
                                             rows_v.at[1 - p], gsem[1 - p])
                        pltpu.make_async_copy(
                            g_hbm.at[src_v.at[j]], rows_v.at[p],
                            gsem[p]).wait()
                        pltpu.sync_copy(rows_v.at[p],
                                        acc.at[dst_c.at[q, jj]], add=True)

                    @pl.when(g + 1 < G)
                    def _():
                        pltpu.make_async_copy(
                            dst_hbm.at[wid, pl.ds(0, GB)],
                            dst_c.at[1 - q], isem[1 - q]).wait()

        plsc.subcore_barrier()
        pltpu.sync_copy(acc.at[pl.ds(base, STRIPE)],
                        out_hbm.at[cid, pl.ds(base, STRIPE)])

    @functools.partial(
        pl.kernel,
        out_type=jax.ShapeDtypeStruct((NC, NP, 128), jnp.float32),
        mesh=mesh,
        scratch_types=[
            pltpu.VMEM((NB, 128), jnp.int32),
            pltpu.VMEM((2, GB, 128), jnp.int32),
            pltpu.VMEM((2, 128, 128), jnp.float32),
            pltpu.VMEM_SHARED((NP, 128), jnp.float32),
            pltpu.SemaphoreType.DMA,
            pltpu.SemaphoreType.DMA,
            pltpu.SemaphoreType.DMA,
            pltpu.SemaphoreType.DMA,
        ],
    )
    def agg2_sc(src_hbm, dst_hbm, g2_hbm, zeros_hbm, out_hbm,
                src_v, dst_c, rows_v, acc, gs0, gs1, is0, is1):
        cid = lax.axis_index("c")
        sid = lax.axis_index("s")
        gsem = (gs0, gs1)
        isem = (is0, is1)
        g_hbm = g2_hbm.at[cid]

        base = sid * STRIPE
        for k in range(STRIPE // 128):
            pltpu.sync_copy(zeros_hbm, acc.at[pl.ds(base + k * 128, 128)])
        rem = STRIPE % 128
        if rem:
            pltpu.sync_copy(zeros_hbm.at[pl.ds(0, rem)],
                            acc.at[pl.ds(base + (STRIPE // 128) * 128, rem)])
        plsc.subcore_barrier()

        def phase(w):
            pltpu.sync_copy(src_hbm.at[w], src_v)
            pltpu.sync_copy(dst_hbm.at[w, pl.ds(0, GB)], dst_c.at[0])
            pltpu.async_copy(g_hbm.at[src_v.at[0]], rows_v.at[0], gsem[0])

            @pl.loop(0, G)
            def _group(g):
                for q in range(2):
                    @pl.when(lax.rem(g, 2) == q)
                    def _():
                        @pl.when(g + 1 < G)
                        def _():
                            pltpu.async_copy(
                                dst_hbm.at[w, pl.ds((g + 1) * GB, GB)],
                                dst_c.at[1 - q], isem[1 - q])
                        for jj in range(GB):
                            p = jj % 2
                            j = g * GB + jj

                            @pl.when(j + 1 < NB)
                            def _():
                                pltpu.async_copy(
                                    g_hbm.at[src_v.at[j + 1]],
                                    rows_v.at[1 - p], gsem[1 - p])
                            pltpu.make_async_copy(
                                g_hbm.at[src_v.at[j]], rows_v.at[p],
                                gsem[p]).wait()
                            pltpu.sync_copy(rows_v.at[p],
                                            acc.at[dst_c.at[q, jj]],
                                            add=True)

                        @pl.when(g + 1 < G)
                        def _():
                            pltpu.make_async_copy(
                                dst_hbm.at[w, pl.ds(0, GB)],
                                dst_c.at[1 - q], isem[1 - q]).wait()

        phase(2 * sid)
        phase(2 * sid + 1)

        plsc.subcore_barrier()
        pltpu.sync_copy(acc.at[pl.ds(base, STRIPE)],
                        out_hbm.at[cid, pl.ds(base, STRIPE)])

    return deg_sc, agg_sc, agg2_sc


def _deg_sc(dst2):
    return _sc_kernels()[0](dst2)


def _agg_sc(src3, dst3, g, zeros128):
    return _sc_kernels()[1](src3, dst3, g, zeros128)


def _agg2_sc(src3, dst3, g2, zeros128):
    return _sc_kernels()[2](src3, dst3, g2, zeros128)


_R = NP // 8


def _dot(a, w):
    return lax.dot_general(a, w, (((1,), (0,)), ((), ())),
                           precision=lax.Precision.HIGHEST,
                           preferred_element_type=jnp.float32)


def _dis_body(deg_ref, x_ref, dis_ref, g1_ref, *, blk):
    i = pl.program_id(0)
    deg = jnp.sum(deg_ref[...], axis=1, keepdims=True) + 1.0
    dis = lax.rsqrt(deg)
    row = i * blk + lax.broadcasted_iota(jnp.int32, (blk, 1), 0)
    dis = jnp.where(row < N, dis, 0.0)
    dis_ref[...] = jnp.broadcast_to(dis, (blk, 128))
    g1_ref[...] = dis * x_ref[...]


def _dis_g1(deg_parts, x_pad):
    return pl.pallas_call(
        functools.partial(_dis_body, blk=_R),
        grid=(NP // _R,),
        in_specs=[
            pl.BlockSpec((_R, NW), lambda i: (i, 0)),
            pl.BlockSpec((_R, 128), lambda i: (i, 0)),
        ],
        out_specs=[
            pl.BlockSpec((_R, 128), lambda i: (i, 0)),
            pl.BlockSpec((_R, 128), lambda i: (i, 0)),
        ],
        out_shape=[
            jax.ShapeDtypeStruct((NP, 128), jnp.float32),
            jax.ShapeDtypeStruct((NP, 128), jnp.float32),
        ],
    )(deg_parts, x_pad)


def _l1_body(s_ref, g_ref, dis_ref, w_ref, b_ref, out_ref):
    dis = dis_ref[...]
    a = dis * (s_ref[0] + s_ref[1] + g_ref[...])
    h = jnp.maximum(_dot(a, w_ref[...]) + b_ref[...], 0.0)
    d1 = dis[:, 0:1]
    out_ref[0] = d1 * h[:, :128]
    out_ref[1] = d1 * h[:, 128:]


def _layer1(s1, g1, dis_b, W1, b1):
    return pl.pallas_call(
        _l1_body,
        grid=(NP // _R,),
        in_specs=[
            pl.BlockSpec((2, _R, 128), lambda i: (0, i, 0)),
            pl.BlockSpec((_R, 128), lambda i: (i, 0)),
            pl.BlockSpec((_R, 128), lambda i: (i, 0)),
            pl.BlockSpec((128, 256), lambda i: (0, 0)),
            pl.BlockSpec((1, 256), lambda i: (0, 0)),
        ],
        out_specs=pl.BlockSpec((2, _R, 128), lambda i: (0, i, 0)),
        out_shape=jax.ShapeDtypeStruct((2, NP, 128), jnp.float32),
    )(s1, g1, dis_b, W1, b1)


def _mid_body(s_ref, g_ref, dis_ref, w_ref, b_ref, out_ref, *,
              w2_ref=None):
    dis = dis_ref[...]
    t0 = dis * (s_ref[0] + g_ref[0])
    t1 = dis * (s_ref[1] + g_ref[1])
    a = jnp.concatenate([t0, t1], axis=1)
    h = jnp.maximum(_dot(a, w_ref[...]) + b_ref[...], 0.0)
    d1 = dis[:, 0:1]
    if w2_ref is None:
        out_ref[0] = d1 * h[:, :128]
        out_ref[1] = d1 * h[:, 128:]
    else:
        out_ref[...] = d1 * _dot(h, w2_ref[...])


def _layer_mid(s2, g, dis_b, W, b):
    return pl.pallas_call(
        _mid_body,
        grid=(NP // _R,),
        in_specs=[
            pl.BlockSpec((2, _R, 128), lambda i: (0, i, 0)),
            pl.BlockSpec((2, _R, 128), lambda i: (0, i, 0)),
            pl.BlockSpec((_R, 128), lambda i: (i, 0)),
            pl.BlockSpec((256, 256), lambda i: (0, 0)),
            pl.BlockSpec((1, 256), lambda i: (0, 0)),
        ],
        out_specs=pl.BlockSpec((2, _R, 128), lambda i: (0, i, 0)),
        out_shape=jax.ShapeDtypeStruct((2, NP, 128), jnp.float32),
    )(s2, g, dis_b, W, b)


def _l3_body(s_ref, g_ref, dis_ref, w_ref, b_ref, w2_ref, out_ref):
    _mid_body(s_ref, g_ref, dis_ref, w_ref, b_ref, out_ref, w2_ref=w2_ref)


def _layer3(s2, g, dis_b, W3, b3, W4):
    return pl.pallas_call(
        _l3_body,
        grid=(NP // _R,),
        in_specs=[
            pl.BlockSpec((2, _R, 128), lambda i: (0, i, 0)),
            pl.BlockSpec((2, _R, 128), lambda i: (0, i, 0)),
            pl.BlockSpec((_R, 128), lambda i: (i, 0)),
            pl.BlockSpec((256, 256), lambda i: (0, 0)),
            pl.BlockSpec((1, 256), lambda i: (0, 0)),
            pl.BlockSpec((256, 128), lambda i: (0, 0)),
        ],
        out_specs=pl.BlockSpec((_R, 128), lambda i: (i, 0)),
        out_shape=jax.ShapeDtypeStruct((NP, 128), jnp.float32),
    )(s2, g, dis_b, W3, b3, W4)


def _l4_body(s_ref, g_ref, dis_ref, b_ref, out_ref):
    z = dis_ref[...] * (s_ref[0] + s_ref[1] + g_ref[...]) + b_ref[...]
    m = jnp.max(z, axis=1, keepdims=True)
    zm = z - m
    out_ref[...] = zm - jnp.log(jnp.sum(jnp.exp(zm), axis=1, keepdims=True))


def _layer4(s4, g4, dis_b, b4):
    blk = 400
    return pl.pallas_call(
        _l4_body,
        grid=(N // blk,),
        in_specs=[
            pl.BlockSpec((2, blk, 128), lambda i: (0, i, 0)),
            pl.BlockSpec((blk, 128), lambda i: (i, 0)),
            pl.BlockSpec((blk, 128), lambda i: (i, 0)),
            pl.BlockSpec((1, 128), lambda i: (0, 0)),
        ],
        out_specs=pl.BlockSpec((blk, 128), lambda i: (i, 0)),
        out_shape=jax.ShapeDtypeStruct((N, 128), jnp.float32),
    )(s4, g4, dis_b, b4)


def kernel(x, edge_index, W1, b1, W2, b2, W3, b3, W4, b4):
    src = edge_index[0].astype(jnp.int32)
    dst = edge_index[1].astype(jnp.int32)
    pad = N + jnp.arange(EP - E, dtype=jnp.int32) % (NP - N)
    src3 = jnp.concatenate([src, pad]).reshape(NW, NB, 128)
    dst3 = jnp.concatenate([dst, pad]).reshape(NW, NB, 128)
    dst2 = dst3.reshape(NW, EPT)
    x_pad = jnp.pad(x, ((0, NP - N), (0, 0)))
    zeros128 = jnp.zeros((128, 128), jnp.float32)

    agg = lambda g: _agg_sc(src3, dst3, g, zeros128)

    b1r, b2r, b3r, b4r = (b.reshape(1, -1) for b in (b1, b2, b3, b4))
    deg_parts = _deg_sc(dst2)
    dis_b, g1 = _dis_g1(deg_parts.T, x_pad)
    s1 = agg(g1)
    g2 = _layer1(s1, g1, dis_b, W1, b1r)
    s2 = _agg2_sc(src3, dst3, g2, zeros128)
    g3 = _layer_mid(s2, g2, dis_b, W2, b2r)
    s3 = _agg2_sc(src3, dst3, g3, zeros128)
    g4 = _layer3(s3, g3, dis_b, W3, b3r, W4)
    s4 = agg(g4)
    return _layer4(s4, g4, dis_b, b4r)


# speedup vs baseline: 4.1170x; 1.0335x over previous
import functools

import jax
import jax.numpy as jnp
from jax import lax
from jax.experimental import pallas as pl
from jax.experimental.pallas import tpu as pltpu
from jax.experimental.pallas import tpu_sc as plsc

N = 10000
E = 320000
NP = 10112
NC, NS = 2, 16
NW = NC * NS
EPT = 10240
EP = EPT * NW
NB = EPT // 128
GB = 8
G = NB // GB
STRIPE = NP // NS

@functools.cache
def _sc_kernels():
    mesh = plsc.VectorSubcoreMesh(core_axis_name="c", subcore_axis_name="s",
                                  num_cores=NC, num_subcores=NS)

    @functools.partial(
        pl.kernel,
        out_type=jax.ShapeDtypeStruct((NW, NP), jnp.float32),
        mesh=mesh,
        scratch_types=[
            pltpu.VMEM((EPT,), jnp.int32),
            pltpu.VMEM((NP,), jnp.float32),
        ],
        compiler_params=pltpu.CompilerParams(needs_layout_passes=False),
    )
    def deg_sc(dst_hbm, deg_out, dst_v, cnt_v):
        cid = lax.axis_index("c")
        sid = lax.axis_index("s")
        wid = sid * NC + cid
        pltpu.sync_copy(dst_hbm.at[wid], dst_v)

        @pl.loop(0, NP // 16)
        def _zero(i):
            cnt_v[pl.ds(i * 16, 16)] = jnp.zeros((16,), jnp.float32)

        ones = jnp.ones((16,), jnp.float32)

        @pl.loop(0, EPT // 16)
        def _count(i):
            idx = dst_v[pl.ds(i * 16, 16)]
            plsc.addupdate_scatter(cnt_v, [idx], ones)

        pltpu.sync_copy(cnt_v, deg_out.at[wid])

    @functools.partial(
        pl.kernel,
        out_type=jax.ShapeDtypeStruct((NC, NP, 128), jnp.float32),
        mesh=mesh,
        scratch_types=[
            pltpu.VMEM((NB, 128), jnp.int32),
            pltpu.VMEM((2, GB, 128), jnp.int32),
            pltpu.VMEM((2, 128, 128), jnp.float32),
            pltpu.VMEM_SHARED((NP, 128), jnp.float32),
            pltpu.SemaphoreType.DMA,
            pltpu.SemaphoreType.DMA,
            pltpu.SemaphoreType.DMA,
            pltpu.SemaphoreType.DMA,
        ],
    )
    def agg_sc(src_hbm, dst_hbm, g_hbm, zeros_hbm, out_hbm,
               src_v, dst_c, rows_v, acc, gs0, gs1, is0, is1):
        cid = lax.axis_index("c")
        sid = lax.axis_index("s")
        wid = sid * NC + cid
        gsem = (gs0, gs1)
        isem = (is0, is1)

        base = sid * STRIPE
        for k in range(STRIPE // 128):
            pltpu.sync_copy(zeros_hbm, acc.at[pl.ds(base + k * 128, 128)])
        rem = STRIPE % 128
        if rem:
            pltpu.sync_copy(zeros_hbm.at[pl.ds(0, rem)],
                            acc.at[pl.ds(base + (STRIPE // 128) * 128, rem)])

        pltpu.sync_copy(src_hbm.at[wid], src_v)
        pltpu.sync_copy(dst_hbm.at[wid, pl.ds(0, GB)], dst_c.at[0])
        plsc.subcore_barrier()
        pltpu.async_copy(g_hbm.at[src_v.at[0]], rows_v.at[0], gsem[0])

        @pl.loop(0, G)
        def _group(g):
            for q in range(2):
                @pl.when(lax.rem(g, 2) == q)
                def _():
                    @pl.when(g + 1 < G)
                    def _():
                        pltpu.async_copy(
                            dst_hbm.at[wid, pl.ds((g + 1) * GB, GB)],
                            dst_c.at[1 - q], isem[1 - q])
                    for jj in range(GB):
                        p = jj % 2
                        j = g * GB + jj

                        @pl.when(j + 1 < NB)
                        def _():
                            pltpu.async_copy(g_hbm.at[src_v.at[j + 1]],
                                             rows_v.at[1 - p], gsem[1 - p])
                        pltpu.make_async_copy(
                            g_hbm.at[src_v.at[j]], rows_v.at[p],
                            gsem[p]).wait()
                        pltpu.sync_copy(rows_v.at[p],
                                        acc.at[dst_c.at[q, jj]], add=True)

                    @pl.when(g + 1 < G)
                    def _():
                        pltpu.make_async_copy(
                            dst_hbm.at[wid, pl.ds(0, GB)],
                            dst_c.at[1 - q], isem[1 - q]).wait()

        plsc.subcore_barrier()
        pltpu.sync_copy(acc.at[pl.ds(base, STRIPE)],
                        out_hbm.at[cid, pl.ds(base, STRIPE)])

    @functools.partial(
        pl.kernel,
        out_type=jax.ShapeDtypeStruct((NC, NP, 128), jnp.float32),
        mesh=mesh,
        scratch_types=[
            pltpu.VMEM((NB, 128), jnp.int32),
            pltpu.VMEM((2, GB, 128), jnp.int32),
            pltpu.VMEM((2, 128, 128), jnp.float32),
            pltpu.VMEM_SHARED((NP, 128), jnp.float32),
            pltpu.SemaphoreType.DMA,
            pltpu.SemaphoreType.DMA,
            pltpu.SemaphoreType.DMA,
            pltpu.SemaphoreType.DMA,
        ],
    )
    def agg2_sc(src_hbm, dst_hbm, g2_hbm, zeros_hbm, out_hbm,
                src_v, dst_c, rows_v, acc, gs0, gs1, is0, is1):
        cid = lax.axis_index("c")
        sid = lax.axis_index("s")
        gsem = (gs0, gs1)
        isem = (is0, is1)
        g_hbm = g2_hbm.at[cid]

        base = sid * STRIPE
        for k in range(STRIPE // 128):
            pltpu.sync_copy(zeros_hbm, acc.at[pl.ds(base + k * 128, 128)])
        rem = STRIPE % 128
        if rem:
            pltpu.sync_copy(zeros_hbm.at[pl.ds(0, rem)],
                            acc.at[pl.ds(base + (STRIPE // 128) * 128, rem)])
        plsc.subcore_barrier()

        def phase(w):
            pltpu.sync_copy(src_hbm.at[w], src_v)
            pltpu.sync_copy(dst_hbm.at[w, pl.ds(0, GB)], dst_c.at[0])
            pltpu.async_copy(g_hbm.at[src_v.at[0]], rows_v.at[0], gsem[0])

            @pl.loop(0, G)
            def _group(g):
                for q in range(2):
                    @pl.when(lax.rem(g, 2) == q)
                    def _():
                        @pl.when(g + 1 < G)
                        def _():
                            pltpu.async_copy(
                                dst_hbm.at[w, pl.ds((g + 1) * GB, GB)],
                                dst_c.at[1 - q], isem[1 - q])
                        for jj in range(GB):
                            p = jj % 2
                            j = g * GB + jj

                            @pl.when(j + 1 < NB)
                            def _():
                                pltpu.async_copy(
                                    g_hbm.at[src_v.at[j + 1]],
                                    rows_v.at[1 - p], gsem[1 - p])
                            pltpu.make_async_copy(
                                g_hbm.at[src_v.at[j]], rows_v.at[p],
                                gsem[p]).wait()
                            pltpu.sync_copy(rows_v.at[p],
                                            acc.at[dst_c.at[q, jj]],
                                            add=True)

                        @pl.when(g + 1 < G)
                        def _():
                            pltpu.make_async_copy(
                                dst_hbm.at[w, pl.ds(0, GB)],
                                dst_c.at[1 - q], isem[1 - q]).wait()

        phase(2 * sid)
        phase(2 * sid + 1)

        plsc.subcore_barrier()
        pltpu.sync_copy(acc.at[pl.ds(base, STRIPE)],
                        out_hbm.at[cid, pl.ds(base, STRIPE)])

    return deg_sc, agg_sc, agg2_sc


def _deg_sc(dst2):
    return _sc_kernels()[0](dst2)


def _agg_sc(src3, dst3, g, zeros128):
    return _sc_kernels()[1](src3, dst3, g, zeros128)


def _agg2_sc(src3, dst3, g2, zeros128):
    return _sc_kernels()[2](src3, dst3, g2, zeros128)


_R = NP // 8


def _dot(a, w):
    return lax.dot_general(a, w, (((1,), (0,)), ((), ())),
                           preferred_element_type=jnp.float32)


def _dis_body(deg_ref, x_ref, dis_ref, g1_ref, *, blk):
    i = pl.program_id(0)
    deg = jnp.sum(deg_ref[...], axis=1, keepdims=True) + 1.0
    dis = lax.rsqrt(deg)
    row = i * blk + lax.broadcasted_iota(jnp.int32, (blk, 1), 0)
    dis = jnp.where(row < N, dis, 0.0)
    dis_ref[...] = jnp.broadcast_to(dis, (blk, 128))
    g1_ref[...] = dis * x_ref[...]


def _dis_g1(deg_parts, x_pad):
    return pl.pallas_call(
        functools.partial(_dis_body, blk=_R),
        grid=(NP // _R,),
        in_specs=[
            pl.BlockSpec((_R, NW), lambda i: (i, 0)),
            pl.BlockSpec((_R, 128), lambda i: (i, 0)),
        ],
        out_specs=[
            pl.BlockSpec((_R, 128), lambda i: (i, 0)),
            pl.BlockSpec((_R, 128), lambda i: (i, 0)),
        ],
        out_shape=[
            jax.ShapeDtypeStruct((NP, 128), jnp.float32),
            jax.ShapeDtypeStruct((NP, 128), jnp.float32),
        ],
    )(deg_parts, x_pad)


def _l1_body(s_ref, g_ref, dis_ref, w_ref, b_ref, out_ref):
    dis = dis_ref[...]
    a = dis * (s_ref[0] + s_ref[1] + g_ref[...])
    h = jnp.maximum(_dot(a, w_ref[...]) + b_ref[...], 0.0)
    d1 = dis[:, 0:1]
    out_ref[0] = d1 * h[:, :128]
    out_ref[1] = d1 * h[:, 128:]


def _layer1(s1, g1, dis_b, W1, b1):
    return pl.pallas_call(
        _l1_body,
        grid=(NP // _R,),
        in_specs=[
            pl.BlockSpec((2, _R, 128), lambda i: (0, i, 0)),
            pl.BlockSpec((_R, 128), lambda i: (i, 0)),
            pl.BlockSpec((_R, 128), lambda i: (i, 0)),
            pl.BlockSpec((128, 256), lambda i: (0, 0)),
            pl.BlockSpec((1, 256), lambda i: (0, 0)),
        ],
        out_specs=pl.BlockSpec((2, _R, 128), lambda i: (0, i, 0)),
        out_shape=jax.ShapeDtypeStruct((2, NP, 128), jnp.float32),
    )(s1, g1, dis_b, W1, b1)


def _mid_body(s_ref, g_ref, dis_ref, w_ref, b_ref, out_ref, *,
              w2_ref=None):
    dis = dis_ref[...]
    t0 = dis * (s_ref[0] + g_ref[0])
    t1 = dis * (s_ref[1] + g_ref[1])
    a = jnp.concatenate([t0, t1], axis=1)
    h = jnp.maximum(_dot(a, w_ref[...]) + b_ref[...], 0.0)
    d1 = dis[:, 0:1]
    if w2_ref is None:
        out_ref[0] = d1 * h[:, :128]
        out_ref[1] = d1 * h[:, 128:]
    else:
        out_ref[...] = d1 * _dot(h, w2_ref[...])


def _layer_mid(s2, g, dis_b, W, b):
    return pl.pallas_call(
        _mid_body,
        grid=(NP // _R,),
        in_specs=[
            pl.BlockSpec((2, _R, 128), lambda i: (0, i, 0)),
            pl.BlockSpec((2, _R, 128), lambda i: (0, i, 0)),
            pl.BlockSpec((_R, 128), lambda i: (i, 0)),
            pl.BlockSpec((256, 256), lambda i: (0, 0)),
            pl.BlockSpec((1, 256), lambda i: (0, 0)),
        ],
        out_specs=pl.BlockSpec((2, _R, 128), lambda i: (0, i, 0)),
        out_shape=jax.ShapeDtypeStruct((2, NP, 128), jnp.float32),
    )(s2, g, dis_b, W, b)


def _l3_body(s_ref, g_ref, dis_ref, w_ref, b_ref, w2_ref, out_ref):
    _mid_body(s_ref, g_ref, dis_ref, w_ref, b_ref, out_ref, w2_ref=w2_ref)


def _layer3(s2, g, dis_b, W3, b3, W4):
    return pl.pallas_call(
        _l3_body,
        grid=(NP // _R,),
        in_specs=[
            pl.BlockSpec((2, _R, 128), lambda i: (0, i, 0)),
            pl.BlockSpec((2, _R, 128), lambda i: (0, i, 0)),
            pl.BlockSpec((_R, 128), lambda i: (i, 0)),
            pl.BlockSpec((256, 256), lambda i: (0, 0)),
            pl.BlockSpec((1, 256), lambda i: (0, 0)),
            pl.BlockSpec((256, 128), lambda i: (0, 0)),
        ],
        out_specs=pl.BlockSpec((_R, 128), lambda i: (i, 0)),
        out_shape=jax.ShapeDtypeStruct((NP, 128), jnp.float32),
    )(s2, g, dis_b, W3, b3, W4)


def _l4_body(s_ref, g_ref, dis_ref, b_ref, out_ref):
    z = dis_ref[...] * (s_ref[0] + s_ref[1] + g_ref[...]) + b_ref[...]
    m = jnp.max(z, axis=1, keepdims=True)
    zm = z - m
    out_ref[...] = zm - jnp.log(jnp.sum(jnp.exp(zm), axis=1, keepdims=True))


def _layer4(s4, g4, dis_b, b4):
    blk = 400
    return pl.pallas_call(
        _l4_body,
        grid=(N // blk,),
        in_specs=[
            pl.BlockSpec((2, blk, 128), lambda i: (0, i, 0)),
            pl.BlockSpec((blk, 128), lambda i: (i, 0)),
            pl.BlockSpec((blk, 128), lambda i: (i, 0)),
            pl.BlockSpec((1, 128), lambda i: (0, 0)),
        ],
        out_specs=pl.BlockSpec((blk, 128), lambda i: (i, 0)),
        out_shape=jax.ShapeDtypeStruct((N, 128), jnp.float32),
    )(s4, g4, dis_b, b4)


def kernel(x, edge_index, W1, b1, W2, b2, W3, b3, W4, b4):
    src = edge_index[0].astype(jnp.int32)
    dst = edge_index[1].astype(jnp.int32)
    pad = N + jnp.arange(EP - E, dtype=jnp.int32) % (NP - N)
    src3 = jnp.concatenate([src, pad]).reshape(NW, NB, 128)
    dst3 = jnp.concatenate([dst, pad]).reshape(NW, NB, 128)
    dst2 = dst3.reshape(NW, EPT)
    x_pad = jnp.pad(x, ((0, NP - N), (0, 0)))
    zeros128 = jnp.zeros((128, 128), jnp.float32)

    agg = lambda g: _agg_sc(src3, dst3, g, zeros128)

    b1r, b2r, b3r, b4r = (b.reshape(1, -1) for b in (b1, b2, b3, b4))
    deg_parts = _deg_sc(dst2)
    dis_b, g1 = _dis_g1(deg_parts.T, x_pad)
    s1 = agg(g1)
    g2 = _layer1(s1, g1, dis_b, W1, b1r)
    s2 = _agg2_sc(src3, dst3, g2, zeros128)
    g3 = _layer_mid(s2, g2, dis_b, W2, b2r)
    s3 = _agg2_sc(src3, dst3, g3, zeros128)
    g4 = _layer3(s3, g3, dis_b, W3, b3r, W4)
    s4 = agg(g4)
    return _layer4(s4, g4, dis_b, b4r)


# speedup vs baseline: 4.1519x; 1.0085x over previous
import functools

import jax
import jax.numpy as jnp
from jax import lax
from jax.experimental import pallas as pl
from jax.experimental.pallas import tpu as pltpu
from jax.experimental.pallas import tpu_sc as plsc

N = 10000
E = 320000
NP = 10112
NC, NS = 2, 16
NW = NC * NS
EPT = 10240
EP = EPT * NW
NB = EPT // 128
GB = 8
G = NB // GB
STRIPE = NP // NS

@functools.cache
def _sc_kernels():
    mesh = plsc.VectorSubcoreMesh(core_axis_name="c", subcore_axis_name="s",
                                  num_cores=NC, num_subcores=NS)

    @functools.partial(
        pl.kernel,
        out_type=jax.ShapeDtypeStruct((NW, NP), jnp.float32),
        mesh=mesh,
        scratch_types=[
            pltpu.VMEM((EPT,), jnp.int32),
            pltpu.VMEM((NP,), jnp.float32),
        ],
        compiler_params=pltpu.CompilerParams(needs_layout_passes=False),
    )
    def deg_sc(dst_hbm, deg_out, dst_v, cnt_v):
        cid = lax.axis_index("c")
        sid = lax.axis_index("s")
        wid = sid * NC + cid
        pltpu.sync_copy(dst_hbm.at[wid], dst_v)

        @pl.loop(0, NP // 16)
        def _zero(i):
            cnt_v[pl.ds(i * 16, 16)] = jnp.zeros((16,), jnp.float32)

        ones = jnp.ones((16,), jnp.float32)

        @pl.loop(0, EPT // 16)
        def _count(i):
            idx = dst_v[pl.ds(i * 16, 16)]
            plsc.addupdate_scatter(cnt_v, [idx], ones)

        pltpu.sync_copy(cnt_v, deg_out.at[wid])

    @functools.partial(
        pl.kernel,
        out_type=jax.ShapeDtypeStruct((NC, NP, 128), jnp.float32),
        mesh=mesh,
        scratch_types=[
            pltpu.VMEM((NB, 128), jnp.int32),
            pltpu.VMEM((2, GB, 128), jnp.int32),
            pltpu.VMEM((2, 128, 128), jnp.float32),
            pltpu.VMEM_SHARED((NP, 128), jnp.float32),
            pltpu.SemaphoreType.DMA,
            pltpu.SemaphoreType.DMA,
            pltpu.SemaphoreType.DMA,
            pltpu.SemaphoreType.DMA,
        ],
    )
    def agg_sc(src_hbm, dst_hbm, g_hbm, zeros_hbm, out_hbm,
               src_v, dst_c, rows_v, acc, gs0, gs1, is0, is1):
        cid = lax.axis_index("c")
        sid = lax.axis_index("s")
        wid = sid * NC + cid
        gsem = (gs0, gs1)
        isem = (is0, is1)

        base = sid * STRIPE
        zdesc = []
        for k in range(STRIPE // 128):
            zdesc.append(pltpu.async_copy(
                zeros_hbm, acc.at[pl.ds(base + k * 128, 128)], isem[0]))
        rem = STRIPE % 128
        if rem:
            zdesc.append(pltpu.async_copy(
                zeros_hbm.at[pl.ds(0, rem)],
                acc.at[pl.ds(base + (STRIPE // 128) * 128, rem)], isem[0]))
        sdesc = pltpu.async_copy(src_hbm.at[wid], src_v, isem[1])
        ddesc = pltpu.async_copy(dst_hbm.at[wid, pl.ds(0, GB)], dst_c.at[0],
                                 gsem[1])
        for d in zdesc:
            d.wait()
        sdesc.wait()
        ddesc.wait()
        plsc.subcore_barrier()
        pltpu.async_copy(g_hbm.at[src_v.at[0]], rows_v.at[0], gsem[0])

        @pl.loop(0, G)
        def _group(g):
            for q in range(2):
                @pl.when(lax.rem(g, 2) == q)
                def _():
                    @pl.when(g + 1 < G)
                    def _():
                        pltpu.async_copy(
                            dst_hbm.at[wid, pl.ds((g + 1) * GB, GB)],
                            dst_c.at[1 - q], isem[1 - q])
                    for jj in range(GB):
                        p = jj % 2
                        j = g * GB + jj

                        @pl.when(j + 1 < NB)
                        def _():
                            pltpu.async_copy(g_hbm.at[src_v.at[j + 1]],
                                             rows_v.at[1 - p], gsem[1 - p])
                        pltpu.make_async_copy(
                            g_hbm.at[src_v.at[j]], rows_v.at[p],
                            gsem[p]).wait()
                        pltpu.sync_copy(rows_v.at[p],
                                        acc.at[dst_c.at[q, jj]], add=True)

                    @pl.when(g + 1 < G)
                    def _():
                        pltpu.make_async_copy(
                            dst_hbm.at[wid, pl.ds(0, GB)],
                            dst_c.at[1 - q], isem[1 - q]).wait()

        plsc.subcore_barrier()
        pltpu.sync_copy(acc.at[pl.ds(base, STRIPE)],
                        out_hbm.at[cid, pl.ds(base, STRIPE)])

    @functools.partial(
        pl.kernel,
        out_type=jax.ShapeDtypeStruct((NC, NP, 128), jnp.float32),
        mesh=mesh,
        scratch_types=[
            pltpu.VMEM((NB, 128), jnp.int32),
            pltpu.VMEM((2, GB, 128), jnp.int32),
            pltpu.VMEM((2, 128, 128), jnp.float32),
            pltpu.VMEM_SHARED((NP, 128), jnp.float32),
            pltpu.SemaphoreType.DMA,
            pltpu.SemaphoreType.DMA,
            pltpu.SemaphoreType.DMA,
            pltpu.SemaphoreType.DMA,
        ],
    )
    def agg2_sc(src_hbm, dst_hbm, g2_hbm, zeros_hbm, out_hbm,
                src_v, dst_c, rows_v, acc, gs0, gs1, is0, is1):
        cid = lax.axis_index("c")
        sid = lax.axis_index("s")
        gsem = (gs0, gs1)
        isem = (is0, is1)
        g_hbm = g2_hbm.at[cid]

        base = sid * STRIPE
        zdesc = []
        for k in range(STRIPE // 128):
            zdesc.append(pltpu.async_copy(
                zeros_hbm, acc.at[pl.ds(base + k * 128, 128)], isem[0]))
        rem = STRIPE % 128
        if rem:
            zdesc.append(pltpu.async_copy(
                zeros_hbm.at[pl.ds(0, rem)],
                acc.at[pl.ds(base + (STRIPE // 128) * 128, rem)], isem[0]))
        for d in zdesc:
            d.wait()
        plsc.subcore_barrier()

        def phase(w):
            sdesc = pltpu.async_copy(src_hbm.at[w], src_v, isem[1])
            ddesc = pltpu.async_copy(dst_hbm.at[w, pl.ds(0, GB)],
                                     dst_c.at[0], gsem[1])
            sdesc.wait()
            ddesc.wait()
            pltpu.async_copy(g_hbm.at[src_v.at[0]], rows_v.at[0], gsem[0])

            @pl.loop(0, G)
            def _group(g):
                for q in range(2):
                    @pl.when(lax.rem(g, 2) == q)
                    def _():
                        @pl.when(g + 1 < G)
                        def _():
                            pltpu.async_copy(
                                dst_hbm.at[w, pl.ds((g + 1) * GB, GB)],
                                dst_c.at[1 - q], isem[1 - q])
                        for jj in range(GB):
                            p = jj % 2
                            j = g * GB + jj

                            @pl.when(j + 1 < NB)
                            def _():
                                pltpu.async_copy(
                                    g_hbm.at[src_v.at[j + 1]],
                                    rows_v.at[1 - p], gsem[1 - p])
                            pltpu.make_async_copy(
                                g_hbm.at[src_v.at[j]], rows_v.at[p],
                                gsem[p]).wait()
                            pltpu.sync_copy(rows_v.at[p],
                                            acc.at[dst_c.at[q, jj]],
                                            add=True)

                        @pl.when(g + 1 < G)
                        def _():
                            pltpu.make_async_copy(
                                dst_hbm.at[w, pl.ds(0, GB)],
                                dst_c.at[1 - q], isem[1 - q]).wait()

        phase(2 * sid)
        phase(2 * sid + 1)

        plsc.subcore_barrier()
        pltpu.sync_copy(acc.at[pl.ds(base, STRIPE)],
                        out_hbm.at[cid, pl.ds(base, STRIPE)])

    return deg_sc, agg_sc, agg2_sc


def _deg_sc(dst2):
    return _sc_kernels()[0](dst2)


def _agg_sc(src3, dst3, g, zeros128):
    return _sc_kernels()[1](src3, dst3, g, zeros128)


def _agg2_sc(src3, dst3, g2, zeros128):
    return _sc_kernels()[2](src3, dst3, g2, zeros128)


_R = NP // 8


def _dot(a, w):
    return lax.dot_general(a, w, (((1,), (0,)), ((), ())),
                           preferred_element_type=jnp.float32)


def _dis_body(deg_ref, x_ref, dis_ref, g1_ref, *, blk):
    i = pl.program_id(0)
    deg = jnp.sum(deg_ref[...], axis=1, keepdims=True) + 1.0
    dis = lax.rsqrt(deg)
    row = i * blk + lax.broadcasted_iota(jnp.int32, (blk, 1), 0)
    dis = jnp.where(row < N, dis, 0.0)
    dis_ref[...] = jnp.broadcast_to(dis, (blk, 128))
    g1_ref[...] = dis * x_ref[...]


def _dis_g1(deg_parts, x_pad):
    return pl.pallas_call(
        functools.partial(_dis_body, blk=_R),
        grid=(NP // _R,),
        in_specs=[
            pl.BlockSpec((_R, NW), lambda i: (i, 0)),
            pl.BlockSpec((_R, 128), lambda i: (i, 0)),
        ],
        out_specs=[
            pl.BlockSpec((_R, 128), lambda i: (i, 0)),
            pl.BlockSpec((_R, 128), lambda i: (i, 0)),
        ],
        out_shape=[
            jax.ShapeDtypeStruct((NP, 128), jnp.float32),
            jax.ShapeDtypeStruct((NP, 128), jnp.float32),
        ],
    )(deg_parts, x_pad)


def _l1_body(s_ref, g_ref, dis_ref, w_ref, b_ref, out_ref):
    dis = dis_ref[...]
    a = dis * (s_ref[0] + s_ref[1] + g_ref[...])
    h = jnp.maximum(_dot(a, w_ref[...]) + b_ref[...], 0.0)
    d1 = dis[:, 0:1]
    out_ref[0] = d1 * h[:, :128]
    out_ref[1] = d1 * h[:, 128:]


def _layer1(s1, g1, dis_b, W1, b1):
    return pl.pallas_call(
        _l1_body,
        grid=(NP // _R,),
        in_specs=[
            pl.BlockSpec((2, _R, 128), lambda i: (0, i, 0)),
            pl.BlockSpec((_R, 128), lambda i: (i, 0)),
            pl.BlockSpec((_R, 128), lambda i: (i, 0)),
            pl.BlockSpec((128, 256), lambda i: (0, 0)),
            pl.BlockSpec((1, 256), lambda i: (0, 0)),
        ],
        out_specs=pl.BlockSpec((2, _R, 128), lambda i: (0, i, 0)),
        out_shape=jax.ShapeDtypeStruct((2, NP, 128), jnp.float32),
    )(s1, g1, dis_b, W1, b1)


def _mid_body(s_ref, g_ref, dis_ref, w_ref, b_ref, out_ref, *,
              w2_ref=None):
    dis = dis_ref[...]
    t0 = dis * (s_ref[0] + g_ref[0])
    t1 = dis * (s_ref[1] + g_ref[1])
    a = jnp.concatenate([t0, t1], axis=1)
    h = jnp.maximum(_dot(a, w_ref[...]) + b_ref[...], 0.0)
    d1 = dis[:, 0:1]
    if w2_ref is None:
        out_ref[0] = d1 * h[:, :128]
        out_ref[1] = d1 * h[:, 128:]
    else:
        out_ref[...] = d1 * _dot(h, w2_ref[...])


def _layer_mid(s2, g, dis_b, W, b):
    return pl.pallas_call(
        _mid_body,
        grid=(NP // _R,),
        in_specs=[
            pl.BlockSpec((2, _R, 128), lambda i: (0, i, 0)),
            pl.BlockSpec((2, _R, 128), lambda i: (0, i, 0)),
            pl.BlockSpec((_R, 128), lambda i: (i, 0)),
            pl.BlockSpec((256, 256), lambda i: (0, 0)),
            pl.BlockSpec((1, 256), lambda i: (0, 0)),
        ],
        out_specs=pl.BlockSpec((2, _R, 128), lambda i: (0, i, 0)),
        out_shape=jax.ShapeDtypeStruct((2, NP, 128), jnp.float32),
    )(s2, g, dis_b, W, b)


def _l3_body(s_ref, g_ref, dis_ref, w_ref, b_ref, w2_ref, out_ref):
    _mid_body(s_ref, g_ref, dis_ref, w_ref, b_ref, out_ref, w2_ref=w2_ref)


def _layer3(s2, g, dis_b, W3, b3, W4):
    return pl.pallas_call(
        _l3_body,
        grid=(NP // _R,),
        in_specs=[
            pl.BlockSpec((2, _R, 128), lambda i: (0, i, 0)),
            pl.BlockSpec((2, _R, 128), lambda i: (0, i, 0)),
            pl.BlockSpec((_R, 128), lambda i: (i, 0)),
            pl.BlockSpec((256, 256), lambda i: (0, 0)),
            pl.BlockSpec((1, 256), lambda i: (0, 0)),
            pl.BlockSpec((256, 128), lambda i: (0, 0)),
        ],
        out_specs=pl.BlockSpec((_R, 128), lambda i: (i, 0)),
        out_shape=jax.ShapeDtypeStruct((NP, 128), jnp.float32),
    )(s2, g, dis_b, W3, b3, W4)


def _l4_body(s_ref, g_ref, dis_ref, b_ref, out_ref):
    z = dis_ref[...] * (s_ref[0] + s_ref[1] + g_ref[...]) + b_ref[...]
    m = jnp.max(z, axis=1, keepdims=True)
    zm = z - m
    out_ref[...] = zm - jnp.log(jnp.sum(jnp.exp(zm), axis=1, keepdims=True))


def _layer4(s4, g4, dis_b, b4):
    blk = 400
    return pl.pallas_call(
        _l4_body,
        grid=(N // blk,),
        in_specs=[
            pl.BlockSpec((2, blk, 128), lambda i: (0, i, 0)),
            pl.BlockSpec((blk, 128), lambda i: (i, 0)),
            pl.BlockSpec((blk, 128), lambda i: (i, 0)),
            pl.BlockSpec((1, 128), lambda i: (0, 0)),
        ],
        out_specs=pl.BlockSpec((blk, 128), lambda i: (i, 0)),
        out_shape=jax.ShapeDtypeStruct((N, 128), jnp.float32),
    )(s4, g4, dis_b, b4)


def kernel(x, edge_index, W1, b1, W2, b2, W3, b3, W4, b4):
    src = edge_index[0].astype(jnp.int32)
    dst = edge_index[1].astype(jnp.int32)
    pad = N + jnp.arange(EP - E, dtype=jnp.int32) % (NP - N)
    src3 = jnp.concatenate([src, pad]).reshape(NW, NB, 128)
    dst3 = jnp.concatenate([dst, pad]).reshape(NW, NB, 128)
    dst2 = dst3.reshape(NW, EPT)
    x_pad = jnp.pad(x, ((0, NP - N), (0, 0)))
    zeros128 = jnp.zeros((128, 128), jnp.float32)

    agg = lambda g: _agg_sc(src3, dst3, g, zeros128)

    b1r, b2r, b3r, b4r = (b.reshape(1, -1) for b in (b1, b2, b3, b4))
    deg_parts = _deg_sc(dst2)
    dis_b, g1 = _dis_g1(deg_parts.T, x_pad)
    s1 = agg(g1)
    g2 = _layer1(s1, g1, dis_b, W1, b1r)
    s2 = _agg2_sc(src3, dst3, g2, zeros128)
    g3 = _layer_mid(s2, g2, dis_b, W2, b2r)
    s3 = _agg2_sc(src3, dst3, g3, zeros128)
    g4 = _layer3(s3, g3, dis_b, W3, b3r, W4)
    s4 = agg(g4)
    return _layer4(s4, g4, dis_b, b4r)
